# Initial kernel scaffold; baseline (speedup 1.0000x reference)
import jax
import jax.numpy as jnp
from jax.experimental import pallas as pl


def kernel(queries, stat_keys, token_keys, values, stat_valid_lens, Wq_stat, Wq_token, Wk_stat, Wk_token, Wv, Wo):
    raise NotImplementedError("write your pallas kernel here")


# speedup vs baseline: 5.5415x; 5.5415x over previous
import functools
import math

import jax
import jax.numpy as jnp
from jax import lax
from jax.experimental import pallas as pl
from jax.experimental.pallas import tpu as pltpu
from jax.experimental.pallas import tpu_sc as plsc

_HP = lax.Precision.HIGHEST

_B, _Q, _S, _T, _D = 8, 4, 64, 512, 128
_BQ = _B * _Q
_STAT_K = 8
_TOK_K = 64
_RSQD = 1.0 / math.sqrt(_D)


def _prep_body(vl_ref, q_ref, sk_ref, wqs_ref, wqt_ref, wks_ref,
               qt_ref, sw_ref, rid_ref):
    q2 = q_ref[...]
    qs = lax.dot(q2, wqs_ref[...])
    qt = lax.dot(q2, wqt_ref[...])
    ks = lax.dot(sk_ref[...], wks_ref[...])
    scores = lax.dot_general(qs, ks, (((1,), (1,)), ((), ()))) * _RSQD
    col = lax.broadcasted_iota(jnp.int32, (_BQ, _B * _S), 1)
    row = lax.broadcasted_iota(jnp.int32, (_BQ, _B * _S), 0)
    own = (col // _S) == (row // _Q)
    vlrow = jnp.zeros((_BQ, _B * _S), jnp.int32)
    for b in range(_B):
        vlrow = jnp.where(row // _Q == b, vl_ref[0, b], vlrow)
    valid = (col % _S) < vlrow
    scores = jnp.where(own & valid, scores,
                       jnp.where(own, -1000000.0, -1e30))
    cur = scores
    vals, idxs = [], []
    for _ in range(_STAT_K):
        m = jnp.max(cur, axis=1, keepdims=True)
        i = jnp.min(jnp.where(cur == m, col, 1 << 30), axis=1, keepdims=True)
        vals.append(m)
        idxs.append(i)
        cur = jnp.where(col == i, -3e30, cur)
    vals = jnp.concatenate(vals, axis=1)
    idxs = jnp.concatenate(idxs, axis=1)
    e = jnp.exp(vals - vals[:, :1])
    sw = e / jnp.sum(e, axis=1, keepdims=True)
    qt_ref[...] = qt
    sw_ref[...] = jnp.concatenate(
        [sw, jnp.zeros((_BQ, _D - _STAT_K), jnp.float32)], axis=1)
    rid_ref[...] = jnp.concatenate(
        [idxs, jnp.zeros((_BQ, _D - _STAT_K), jnp.int32)], axis=1)


def _tscore_body(tkb_ref, wkt_ref, qt_ref, o_ref):
    ktok = lax.dot(tkb_ref[0], wkt_ref[...])
    o_ref[0] = lax.dot_general(qt_ref[0], ktok,
                               (((1,), (1,)), ((), ()))) * _RSQD


def _fin_body(acc_ref, wv_ref, wo_ref, out_ref):
    out_ref[...] = lax.dot(lax.dot(acc_ref[...], wv_ref[...], precision=_HP),
                           wo_ref[...], precision=_HP)


_NC = 2
_NS = 16


def _sc_body(ts_hbm, va_hbm, sw_hbm, rid_hbm, out_hbm,
             sw_v, rid_v, srow, kbuf, selid, selw, vrows, outb, sem):
    wid = lax.axis_index("s") * _NC + lax.axis_index("c")
    qidx = wid % _Q
    pltpu.sync_copy(sw_hbm.at[wid], sw_v)
    pltpu.sync_copy(rid_hbm.at[wid], rid_v)
    lanes = lax.iota(jnp.int32, 16)

    def jbody(j, acc):
        jf = jnp.full((16,), j, jnp.int32)
        bsv = plsc.load_gather(rid_v, [jf])
        bs = jnp.max(bsv)
        pltpu.sync_copy(ts_hbm.at[bs * _Q + qidx], srow)

        def keys_g(g, mvec):
            sv = srow[pl.ds(g * 16, 16)]
            u = plsc.bitcast(sv, jnp.int32)
            kbuf[pl.ds(g * 16, 16)] = u ^ ((u >> 31) & 0x7FFFFFFF)
            return jnp.maximum(mvec, sv)

        mvec = lax.fori_loop(0, _T // 16, keys_g,
                             jnp.full((16,), -3e38, jnp.float32))
        big_m = jnp.max(mvec)

        def bsearch(_, lohi):
            lo, hi = lohi
            diff = hi - lo
            mid = lo + (lax.shift_right_logical(diff, 1) + (diff & 1))

            def cbody(c, cnt):
                m = kbuf[pl.ds(c * 16, 16)] >= mid
                return cnt + plsc.all_reduce_population_count(m)

            cnt = lax.fori_loop(0, _T // 16, cbody, jnp.zeros((16,), jnp.int32))
            g = cnt >= _TOK_K
            return (jnp.where(g, mid, lo), jnp.where(g, hi, mid - 1))

        kT, _hi = lax.fori_loop(
            0, 32, bsearch,
            (jnp.full((16,), -(2 ** 31), jnp.int32),
             jnp.full((16,), 2 ** 31 - 1, jnp.int32)))

        mv = jnp.full((16,), big_m, jnp.float32)

        def epass(c, carry):
            off, zacc = carry
            sv = srow[pl.ds(c * 16, 16)]
            msel = kbuf[pl.ds(c * 16, 16)] >= kT
            ev = jnp.where(msel, jnp.exp(sv - mv), 0.0)
            pos = off + plsc.cumsum(msel.astype(jnp.int32)) - 1
            okm = msel & (pos < _TOK_K)
            gid = bsv * _T + c * 16 + lanes
            plsc.store_scatter(selid, [pos], gid, mask=okm)
            plsc.store_scatter(selw, [pos], ev, mask=okm)
            return (off + plsc.all_reduce_population_count(msel), zacc + ev)

        _off, zacc = lax.fori_loop(
            0, _T // 16, epass,
            (jnp.zeros((16,), jnp.int32), jnp.zeros((16,), jnp.float32)))
        z = jnp.sum(zacc)

        pltpu.async_copy(va_hbm.at[selid], vrows, sem).wait()

        def abody(t, acc2):
            wv = plsc.load_gather(selw, [jnp.full((16,), t, jnp.int32)])
            return tuple(acc2[c] + wv * vrows[t, pl.ds(c * 16, 16)]
                         for c in range(8))

        acc2 = lax.fori_loop(0, _TOK_K, abody,
                             tuple(jnp.zeros((16,), jnp.float32)
                                   for _ in range(8)))
        gsc = plsc.load_gather(sw_v, [jf]) / jnp.full((16,), z, jnp.float32)
        return tuple(acc[c] + gsc * acc2[c] for c in range(8))

    acc = lax.fori_loop(0, _STAT_K, jbody,
                        tuple(jnp.zeros((16,), jnp.float32) for _ in range(8)))
    for c in range(8):
        outb[pl.ds(c * 16, 16)] = acc[c]
    pltpu.sync_copy(outb, out_hbm.at[wid])


@functools.cache
def _sc_main():
    return pl.kernel(
        _sc_body,
        out_type=jax.ShapeDtypeStruct((_BQ, _D), jnp.float32),
        mesh=plsc.VectorSubcoreMesh(core_axis_name="c", subcore_axis_name="s",
                                    num_cores=_NC, num_subcores=_NS),
        compiler_params=pltpu.CompilerParams(needs_layout_passes=False),
        scratch_types=[
            pltpu.VMEM((_D,), jnp.float32),
            pltpu.VMEM((_D,), jnp.int32),
            pltpu.VMEM((_T,), jnp.float32),
            pltpu.VMEM((_T,), jnp.int32),
            pltpu.VMEM((_TOK_K,), jnp.int32),
            pltpu.VMEM((_D,), jnp.float32),
            pltpu.VMEM((_TOK_K, _D), jnp.float32),
            pltpu.VMEM((_D,), jnp.float32),
            pltpu.SemaphoreType.DMA,
        ],
    )


def kernel(queries, stat_keys, token_keys, values, stat_valid_lens,
           Wq_stat, Wq_token, Wk_stat, Wk_token, Wv, Wo):
    q2 = queries.reshape(_BQ, _D)
    sk = stat_keys.reshape(_B * _S, _D)
    vaf = values.reshape(_B * _S * _T, _D)
    vl = stat_valid_lens.reshape(1, _B)

    qt, sw, rid = pl.pallas_call(
        _prep_body,
        out_shape=[
            jax.ShapeDtypeStruct((_BQ, _D), jnp.float32),
            jax.ShapeDtypeStruct((_BQ, _D), jnp.float32),
            jax.ShapeDtypeStruct((_BQ, _D), jnp.int32),
        ],
        in_specs=[pl.BlockSpec(memory_space=pltpu.SMEM)] + [pl.BlockSpec()] * 5,
    )(vl, q2, sk, Wq_stat, Wq_token, Wk_stat)

    tscores = pl.pallas_call(
        _tscore_body,
        grid=(_B * _S,),
        in_specs=[
            pl.BlockSpec((1, _T, _D), lambda i: (i, 0, 0)),
            pl.BlockSpec((_D, _D), lambda i: (0, 0)),
            pl.BlockSpec((1, _Q, _D), lambda i: (i // _S, 0, 0)),
        ],
        out_specs=pl.BlockSpec((1, _Q, _T), lambda i: (i, 0, 0)),
        out_shape=jax.ShapeDtypeStruct((_B * _S, _Q, _T), jnp.float32),
    )(token_keys, Wk_token, qt.reshape(_B, _Q, _D))

    acc = _sc_main()(tscores.reshape(_B * _S * _Q, _T), vaf, sw, rid)

    out = pl.pallas_call(
        _fin_body,
        out_shape=jax.ShapeDtypeStruct((_BQ, _D), jnp.float32),
    )(acc, Wv, Wo)
    return out.reshape(_B, _Q, _D)
